# baseline (speedup 1.0000x reference)
import functools

import jax
import jax.numpy as jnp
from jax import lax
from jax.experimental import pallas as pl
from jax.experimental.pallas import tpu as pltpu
from jax.experimental.pallas import tpu_sc as plsc

B, S, H = 8, 512, 512
G, V = 2, 320
D = 128
N = B * S
TB = 512
GRID = N // TB

NUM_WORKERS = 32
TOK_W = N // NUM_WORKERS


def _tc_body(x_ref, w_ref, b_ref, idx0_ref, idx1_ref, perp_ref, acc_ref):
    step = pl.program_id(0)
    x = x_ref[...]
    w = w_ref[...]
    ht = lax.dot_general(w, x, (((1,), (1,)), ((), ())),
                         preferred_element_type=jnp.float32)
    ht = ht + b_ref[...]
    h0 = ht[:V, :]
    h1 = ht[V:, :]
    a0 = jnp.argmax(h0, axis=0).astype(jnp.int32)
    a1 = jnp.argmax(h1, axis=0).astype(jnp.int32)
    idx0_ref[...] = a0
    idx1_ref[...] = a1 + V

    iot = lax.broadcasted_iota(jnp.int32, (V, TB), 0)
    oh0 = (iot == a0[None, :]).astype(jnp.bfloat16)
    oh1 = (iot == a1[None, :]).astype(jnp.bfloat16)
    ones = jnp.ones((TB, 1), jnp.bfloat16)
    c0 = lax.dot_general(oh0, ones, (((1,), (0,)), ((), ())),
                         preferred_element_type=jnp.float32)
    c1 = lax.dot_general(oh1, ones, (((1,), (0,)), ((), ())),
                         preferred_element_type=jnp.float32)

    @pl.when(step == 0)
    def _init():
        acc_ref[...] = jnp.zeros_like(acc_ref)

    acc_ref[:, 0:1] += c0
    acc_ref[:, 1:2] += c1

    @pl.when(step == GRID - 1)
    def _final():
        p = acc_ref[...] * (1.0 / N)
        ent = -jnp.sum(p * jnp.log(p + 1e-7), axis=0,
                       keepdims=True)
        perp_ref[...] = jnp.sum(jnp.exp(ent), axis=1,
                                keepdims=True)


_tc_call = pl.pallas_call(
    _tc_body,
    grid=(GRID,),
    in_specs=[
        pl.BlockSpec((TB, H), lambda i: (i, 0)),
        pl.BlockSpec((G * V, H), lambda i: (0, 0)),
        pl.BlockSpec((G * V, 1), lambda i: (0, 0)),
    ],
    out_specs=[
        pl.BlockSpec((TB,), lambda i: (i,)),
        pl.BlockSpec((TB,), lambda i: (i,)),
        pl.BlockSpec((1, 1), lambda i: (0, 0)),
    ],
    out_shape=[
        jax.ShapeDtypeStruct((N,), jnp.int32),
        jax.ShapeDtypeStruct((N,), jnp.int32),
        jax.ShapeDtypeStruct((1, 1), jnp.float32),
    ],
    scratch_shapes=[pltpu.VMEM((V, 2), jnp.float32)],
)


def _sc_gather_body(table_hbm, idx0_hbm, idx1_hbm, out_hbm,
                    idx0_v, idx1_v, rows0_v, rows1_v, sem0, sem1, sem2, sem3):
    wid = lax.axis_index("s") * 2 + lax.axis_index("c")
    base = wid * TOK_W
    ld0 = pltpu.async_copy(idx0_hbm.at[pl.ds(base, TOK_W)], idx0_v, sem0)
    ld1 = pltpu.async_copy(idx1_hbm.at[pl.ds(base, TOK_W)], idx1_v, sem1)
    ld0.wait()
    g0 = pltpu.async_copy(table_hbm.at[idx0_v], rows0_v, sem2)
    ld1.wait()
    g1 = pltpu.async_copy(table_hbm.at[idx1_v], rows1_v, sem3)
    g0.wait()
    st0 = pltpu.async_copy(rows0_v, out_hbm.at[pl.ds(base, TOK_W), pl.ds(0, D)],
                           sem0)
    g1.wait()
    st1 = pltpu.async_copy(rows1_v, out_hbm.at[pl.ds(base, TOK_W), pl.ds(D, D)],
                           sem1)
    st0.wait()
    st1.wait()


@functools.cache
def _sc_gather():
    return functools.partial(
        pl.kernel,
        out_type=jax.ShapeDtypeStruct((N, G * D), jnp.float32),
        mesh=plsc.VectorSubcoreMesh(core_axis_name="c", subcore_axis_name="s"),
        scratch_types=[
            pltpu.VMEM((TOK_W,), jnp.int32),
            pltpu.VMEM((TOK_W,), jnp.int32),
            pltpu.VMEM((TOK_W, D), jnp.float32),
            pltpu.VMEM((TOK_W, D), jnp.float32),
            pltpu.SemaphoreType.DMA,
            pltpu.SemaphoreType.DMA,
            pltpu.SemaphoreType.DMA,
            pltpu.SemaphoreType.DMA,
        ],
    )(_sc_gather_body)


def kernel(hidden_states, W, b, codevectors):
    hs2 = hidden_states.reshape(N, H)
    b2 = b.reshape(G * V, 1)
    table = codevectors.reshape(G * V, D)
    idx0, idx1, perp = _tc_call(hs2, W, b2)
    cv = _sc_gather()(table, idx0, idx1)
    return cv.reshape(B, S, G * D), perp[0, 0]


# speedup vs baseline: 1.0735x; 1.0735x over previous
import functools

import jax
import jax.numpy as jnp
from jax import lax
from jax.experimental import pallas as pl
from jax.experimental.pallas import tpu as pltpu
from jax.experimental.pallas import tpu_sc as plsc

B, S, H = 8, 512, 512
G, V = 2, 320
D = 128
N = B * S
TB = 1024
GRID = N // TB

NUM_WORKERS = 32
TOK_W = N // NUM_WORKERS


def _tc_body(x_ref, w_ref, b_ref, idx0_ref, idx1_ref, perp_ref, acc_ref):
    step = pl.program_id(0)
    x = x_ref[...]
    w = w_ref[...]
    ht = lax.dot_general(w, x, (((1,), (1,)), ((), ())),
                         preferred_element_type=jnp.float32)
    ht = ht + b_ref[...]
    h0 = ht[:V, :]
    h1 = ht[V:, :]
    a0 = jnp.argmax(h0, axis=0).astype(jnp.int32)
    a1 = jnp.argmax(h1, axis=0).astype(jnp.int32)
    idx0_ref[...] = a0
    idx1_ref[...] = a1 + V

    iot = lax.broadcasted_iota(jnp.int32, (V, TB), 0)
    oh0 = (iot == a0[None, :]).astype(jnp.bfloat16)
    oh1 = (iot == a1[None, :]).astype(jnp.bfloat16)
    ones = jnp.ones((TB, 1), jnp.bfloat16)
    c0 = lax.dot_general(oh0, ones, (((1,), (0,)), ((), ())),
                         preferred_element_type=jnp.float32)
    c1 = lax.dot_general(oh1, ones, (((1,), (0,)), ((), ())),
                         preferred_element_type=jnp.float32)

    @pl.when(step == 0)
    def _init():
        acc_ref[...] = jnp.zeros_like(acc_ref)

    acc_ref[:, 0:1] += c0
    acc_ref[:, 1:2] += c1

    @pl.when(step == GRID - 1)
    def _final():
        p = acc_ref[...] * (1.0 / N)
        ent = -jnp.sum(p * jnp.log(p + 1e-7), axis=0,
                       keepdims=True)
        perp_ref[...] = jnp.sum(jnp.exp(ent), axis=1,
                                keepdims=True)


_tc_call = pl.pallas_call(
    _tc_body,
    grid=(GRID,),
    in_specs=[
        pl.BlockSpec((TB, H), lambda i: (i, 0)),
        pl.BlockSpec((G * V, H), lambda i: (0, 0)),
        pl.BlockSpec((G * V, 1), lambda i: (0, 0)),
    ],
    out_specs=[
        pl.BlockSpec((TB,), lambda i: (i,)),
        pl.BlockSpec((TB,), lambda i: (i,)),
        pl.BlockSpec((1, 1), lambda i: (0, 0)),
    ],
    out_shape=[
        jax.ShapeDtypeStruct((N,), jnp.int32),
        jax.ShapeDtypeStruct((N,), jnp.int32),
        jax.ShapeDtypeStruct((1, 1), jnp.float32),
    ],
    scratch_shapes=[pltpu.VMEM((V, 2), jnp.float32)],
)


def _sc_gather_body(table_hbm, idx0_hbm, idx1_hbm, out_hbm,
                    idx0_v, idx1_v, rows0_v, rows1_v, sem0, sem1, sem2, sem3):
    wid = lax.axis_index("s") * 2 + lax.axis_index("c")
    base = wid * TOK_W
    ld0 = pltpu.async_copy(idx0_hbm.at[pl.ds(base, TOK_W)], idx0_v, sem0)
    ld1 = pltpu.async_copy(idx1_hbm.at[pl.ds(base, TOK_W)], idx1_v, sem1)
    ld0.wait()
    g0 = pltpu.async_copy(table_hbm.at[idx0_v], rows0_v, sem2)
    ld1.wait()
    g1 = pltpu.async_copy(table_hbm.at[idx1_v], rows1_v, sem3)
    g0.wait()
    st0 = pltpu.async_copy(rows0_v, out_hbm.at[pl.ds(base, TOK_W), pl.ds(0, D)],
                           sem0)
    g1.wait()
    st1 = pltpu.async_copy(rows1_v, out_hbm.at[pl.ds(base, TOK_W), pl.ds(D, D)],
                           sem1)
    st0.wait()
    st1.wait()


@functools.cache
def _sc_gather():
    return functools.partial(
        pl.kernel,
        out_type=jax.ShapeDtypeStruct((N, G * D), jnp.float32),
        mesh=plsc.VectorSubcoreMesh(core_axis_name="c", subcore_axis_name="s"),
        scratch_types=[
            pltpu.VMEM((TOK_W,), jnp.int32),
            pltpu.VMEM((TOK_W,), jnp.int32),
            pltpu.VMEM((TOK_W, D), jnp.float32),
            pltpu.VMEM((TOK_W, D), jnp.float32),
            pltpu.SemaphoreType.DMA,
            pltpu.SemaphoreType.DMA,
            pltpu.SemaphoreType.DMA,
            pltpu.SemaphoreType.DMA,
        ],
    )(_sc_gather_body)


def kernel(hidden_states, W, b, codevectors):
    hs2 = hidden_states.reshape(N, H)
    b2 = b.reshape(G * V, 1)
    table = codevectors.reshape(G * V, D)
    idx0, idx1, perp = _tc_call(hs2, W, b2)
    cv = _sc_gather()(table, idx0, idx1)
    return cv.reshape(B, S, G * D), perp[0, 0]
